# baseline (speedup 1.0000x reference)
import functools

import jax
import jax.numpy as jnp
from jax import lax
from jax.experimental import pallas as pl
from jax.experimental.pallas import tpu as pltpu
from jax.experimental.pallas import tpu_sc as plsc

N = 10000
E = 320000
D = 128
H = 128
G = 64

_INV = 0.9999950000374996

_EDGE_BLK = 128
_NBLOCKS = E // _EDGE_BLK
_NTILES = 32
_BLK_FULL = 80
_BLK_LAST = _NBLOCKS - 31 * _BLK_FULL
_HALF = _BLK_FULL // 2
_NBUF = 2

_RPS = 640
_LAST_ROWS = N - 15 * _RPS

_TC_BLK = 400
_TC_GRID = N // _TC_BLK


def _sc_aggregate(h, src2d, dst2d, zeros_blk):
    mesh = plsc.VectorSubcoreMesh(core_axis_name="c", subcore_axis_name="s")

    @functools.partial(
        pl.kernel,
        mesh=mesh,
        out_type=(jax.ShapeDtypeStruct((N, D), jnp.float32),
                  jax.ShapeDtypeStruct((N, D), jnp.float32)),
        scratch_types=[
            pltpu.VMEM((_HALF, _EDGE_BLK), jnp.int32),
            pltpu.VMEM((_HALF, _EDGE_BLK), jnp.int32),
            pltpu.VMEM((_EDGE_BLK, D), jnp.float32),
            pltpu.VMEM((_EDGE_BLK, D), jnp.float32),
            pltpu.VMEM_SHARED((N, D), jnp.float32),
            pltpu.SemaphoreType.DMA,
            pltpu.SemaphoreType.DMA,
        ],
    )
    def k(h_hbm, src_hbm, dst_hbm, zeros_hbm, out0_hbm, out1_hbm,
          src_v, dst_v, rows_v0, rows_v1, acc_sh, gsem0, gsem1):
        rows_b = (rows_v0, rows_v1)
        gsem_b = (gsem0, gsem1)
        c = lax.axis_index("c")
        s = lax.axis_index("s")
        wid = c * 16 + s
        is_last = wid == _NTILES - 1

        @pl.when(s < 15)
        def _():
            pltpu.sync_copy(zeros_hbm, acc_sh.at[pl.ds(s * _RPS, _RPS)])

        @pl.when(s == 15)
        def _():
            pltpu.sync_copy(zeros_hbm.at[pl.ds(0, _LAST_ROWS)],
                            acc_sh.at[pl.ds(15 * _RPS, _LAST_ROWS)])

        plsc.subcore_barrier()

        def start_gather(j, b):
            pltpu.async_copy(h_hbm.at[src_v.at[j]], rows_b[b], gsem_b[b])

        def wait_gather(j, b):
            pltpu.make_async_copy(h_hbm.at[src_v.at[j]], rows_b[b],
                                  gsem_b[b]).wait()

        def run_pipeline(n_iters, fire_limit):
            for b in range(_NBUF):
                start_gather(b, b)

            def body(g, carry):
                for b in range(_NBUF):
                    j = g * _NBUF + b
                    wait_gather(j, b)
                    pltpu.sync_copy(rows_b[b], acc_sh.at[dst_v.at[j]],
                                    add=True)

                    @pl.when(j + _NBUF < fire_limit)
                    def _():
                        start_gather(j + _NBUF, b)
                return carry

            lax.fori_loop(0, n_iters, body, 0)

        base_blk = wid * _BLK_FULL

        @pl.when(is_last)
        def _():
            pltpu.sync_copy(src_hbm.at[pl.ds(31 * _BLK_FULL, _BLK_LAST)],
                            src_v.at[pl.ds(0, _BLK_LAST)])
            pltpu.sync_copy(dst_hbm.at[pl.ds(31 * _BLK_FULL, _BLK_LAST)],
                            dst_v.at[pl.ds(0, _BLK_LAST)])

        @pl.when(jnp.logical_not(is_last))
        def _():
            pltpu.sync_copy(src_hbm.at[pl.ds(base_blk, _HALF)], src_v)
            pltpu.sync_copy(dst_hbm.at[pl.ds(base_blk, _HALF)], dst_v)

        n0 = jnp.where(is_last, _BLK_LAST // _NBUF, _HALF // _NBUF)
        lim0 = jnp.where(is_last, _BLK_LAST, _HALF)
        run_pipeline(n0, lim0)

        @pl.when(jnp.logical_not(is_last))
        def _():
            pltpu.sync_copy(src_hbm.at[pl.ds(base_blk + _HALF, _HALF)], src_v)
            pltpu.sync_copy(dst_hbm.at[pl.ds(base_blk + _HALF, _HALF)], dst_v)
            run_pipeline(_HALF // _NBUF, _HALF)

        plsc.subcore_barrier()

        out_hbm = (out0_hbm, out1_hbm)
        for ci in range(2):
            @pl.when((c == ci) & (s < 15))
            def _(ci=ci):
                rows = pl.ds(s * _RPS, _RPS)
                pltpu.sync_copy(acc_sh.at[rows], out_hbm[ci].at[rows])

            @pl.when((c == ci) & (s == 15))
            def _(ci=ci):
                rows = pl.ds(15 * _RPS, _LAST_ROWS)
                pltpu.sync_copy(acc_sh.at[rows], out_hbm[ci].at[rows])

    return k(h, src2d, dst2d, zeros_blk)


def _tc_layer(h, a0, a1, w1, b1, w2, b2, gamma, beta):
    def body(h_ref, a0_ref, a1_ref, w1_ref, b1_ref, w2_ref, b2_ref,
             g_ref, bt_ref, o_ref):
        hin = h_ref[...] + a0_ref[...] + a1_ref[...]
        z = jax.lax.dot_general(hin, w1_ref[...], (((1,), (0,)), ((), ())),
                                preferred_element_type=jnp.float32)
        z = jnp.maximum(z + b1_ref[...], 0.0)
        o = jax.lax.dot_general(z, w2_ref[...], (((1,), (0,)), ((), ())),
                                preferred_element_type=jnp.float32)
        o = (o + b2_ref[...]) * (g_ref[...] * _INV) + bt_ref[...]
        o_ref[...] = jnp.maximum(o, 0.0)

    row_spec = pl.BlockSpec((_TC_BLK, D), lambda i: (i, 0))
    full = lambda shape: pl.BlockSpec(shape, lambda i: (0,) * len(shape))
    return pl.pallas_call(
        body,
        grid=(_TC_GRID,),
        in_specs=[row_spec, row_spec, row_spec,
                  full((D, H)), full((1, H)), full((H, H)), full((1, H)),
                  full((1, H)), full((1, H))],
        out_specs=row_spec,
        out_shape=jax.ShapeDtypeStruct((N, H), jnp.float32),
    )(h, a0, a1, w1, b1, w2, b2, gamma, beta)


def _tc_pool_head(h, batch3d, fc1_w, fc1_b, fc2_w, fc2_b):
    def body(h_ref, b_ref, w1_ref, b1_ref, w2_ref, b2_ref, o_ref, acc, cnt):
        i = pl.program_id(0)

        @pl.when(i == 0)
        def _():
            acc[...] = jnp.zeros_like(acc)
            cnt[...] = jnp.zeros_like(cnt)

        bm = b_ref[0]
        gids = jax.lax.broadcasted_iota(jnp.int32, (G, _TC_BLK), 0)
        onehot_t = (gids == bm).astype(jnp.float32)
        acc[...] += jax.lax.dot_general(
            onehot_t, h_ref[...], (((1,), (0,)), ((), ())),
            preferred_element_type=jnp.float32)
        cnt[...] += jnp.sum(onehot_t, axis=1, keepdims=True)

        @pl.when(i == _TC_GRID - 1)
        def _():
            pooled = acc[...] / jnp.maximum(cnt[...], 1.0)
            z = jax.lax.dot_general(pooled, w1_ref[...], (((1,), (0,)), ((), ())),
                                    preferred_element_type=jnp.float32)
            z = jnp.maximum(z + b1_ref[...], 0.0)
            o = jax.lax.dot_general(z, w2_ref[...], (((1,), (0,)), ((), ())),
                                    preferred_element_type=jnp.float32)
            o_ref[...] = o + b2_ref[...]

    full = lambda shape: pl.BlockSpec(shape, lambda i: (0,) * len(shape))
    return pl.pallas_call(
        body,
        grid=(_TC_GRID,),
        in_specs=[pl.BlockSpec((_TC_BLK, D), lambda i: (i, 0)),
                  pl.BlockSpec((1, 1, _TC_BLK), lambda i: (i, 0, 0)),
                  full((H, G)), full((1, G)), full((G, 1)), full((1, 1))],
        out_specs=full((G, 1)),
        out_shape=jax.ShapeDtypeStruct((G, 1), jnp.float32),
        scratch_shapes=[pltpu.VMEM((G, D), jnp.float32),
                        pltpu.VMEM((G, 1), jnp.float32)],
        compiler_params=pltpu.CompilerParams(
            dimension_semantics=("arbitrary",)),
    )(h, batch3d, fc1_w, fc1_b, fc2_w, fc2_b)


def kernel(x, edge_index, batch,
           l0_w1, l0_b1, l0_w2, l0_b2, l0_gamma, l0_beta,
           l1_w1, l1_b1, l1_w2, l1_b2, l1_gamma, l1_beta,
           l2_w1, l2_b1, l2_w2, l2_b2, l2_gamma, l2_beta,
           fc1_w, fc1_b, fc2_w, fc2_b):
    src2d = edge_index[0].reshape(_NBLOCKS, _EDGE_BLK)
    dst2d = edge_index[1].reshape(_NBLOCKS, _EDGE_BLK)
    batch3d = batch.reshape(_TC_GRID, 1, _TC_BLK)
    zeros_blk = jnp.zeros((_RPS, D), jnp.float32)

    h = x
    params = [
        (l0_w1, l0_b1, l0_w2, l0_b2, l0_gamma, l0_beta),
        (l1_w1, l1_b1, l1_w2, l1_b2, l1_gamma, l1_beta),
        (l2_w1, l2_b1, l2_w2, l2_b2, l2_gamma, l2_beta),
    ]
    for w1, b1, w2, b2, g, b in params:
        agg0, agg1 = _sc_aggregate(h, src2d, dst2d, zeros_blk)
        h = _tc_layer(h, agg0, agg1, w1, b1.reshape(1, H),
                      w2, b2.reshape(1, H), g.reshape(1, H), b.reshape(1, H))

    out = _tc_pool_head(h, batch3d, fc1_w, fc1_b.reshape(1, G),
                        fc2_w, fc2_b.reshape(1, 1))
    return jnp.squeeze(out, axis=-1)


# speedup vs baseline: 1.0279x; 1.0279x over previous
import functools

import jax
import jax.numpy as jnp
from jax import lax
from jax.experimental import pallas as pl
from jax.experimental.pallas import tpu as pltpu
from jax.experimental.pallas import tpu_sc as plsc

N = 10000
E = 320000
D = 128
H = 128
G = 64

_INV = 0.9999950000374996

_EDGE_BLK = 128
_NBLOCKS = E // _EDGE_BLK
_NTILES = 32
_BLK_FULL = 80
_BLK_LAST = _NBLOCKS - 31 * _BLK_FULL
_HALF = _BLK_FULL // 2
_NBUF = 2

_RPS = 640
_LAST_ROWS = N - 15 * _RPS

_TC_BLK = 400
_TC_GRID = N // _TC_BLK


def _sc_aggregate(h, edge_index, zeros_blk):
    mesh = plsc.VectorSubcoreMesh(core_axis_name="c", subcore_axis_name="s")

    @functools.partial(
        pl.kernel,
        mesh=mesh,
        out_type=(jax.ShapeDtypeStruct((N, D), jnp.float32),
                  jax.ShapeDtypeStruct((N, D), jnp.float32)),
        scratch_types=[
            pltpu.VMEM((_HALF * _EDGE_BLK,), jnp.int32),
            pltpu.VMEM((_HALF * _EDGE_BLK,), jnp.int32),
            pltpu.VMEM((_EDGE_BLK, D), jnp.float32),
            pltpu.VMEM((_EDGE_BLK, D), jnp.float32),
            pltpu.VMEM_SHARED((N, D), jnp.float32),
            pltpu.SemaphoreType.DMA,
            pltpu.SemaphoreType.DMA,
        ],
    )
    def k(h_hbm, ei_hbm, zeros_hbm, out0_hbm, out1_hbm,
          src_v, dst_v, rows_v0, rows_v1, acc_sh, gsem0, gsem1):
        rows_b = (rows_v0, rows_v1)
        gsem_b = (gsem0, gsem1)
        c = lax.axis_index("c")
        s = lax.axis_index("s")
        wid = c * 16 + s
        is_last = wid == _NTILES - 1

        @pl.when(s < 15)
        def _():
            pltpu.sync_copy(zeros_hbm, acc_sh.at[pl.ds(s * _RPS, _RPS)])

        @pl.when(s == 15)
        def _():
            pltpu.sync_copy(zeros_hbm.at[pl.ds(0, _LAST_ROWS)],
                            acc_sh.at[pl.ds(15 * _RPS, _LAST_ROWS)])

        plsc.subcore_barrier()

        def start_gather(j, b):
            pltpu.async_copy(h_hbm.at[src_v.at[pl.ds(j * _EDGE_BLK, _EDGE_BLK)]],
                             rows_b[b], gsem_b[b])

        def wait_gather(j, b):
            pltpu.make_async_copy(
                h_hbm.at[src_v.at[pl.ds(j * _EDGE_BLK, _EDGE_BLK)]],
                rows_b[b], gsem_b[b]).wait()

        def run_pipeline(n_iters, fire_limit):
            for b in range(_NBUF):
                start_gather(b, b)

            def body(g, carry):
                for b in range(_NBUF):
                    j = g * _NBUF + b
                    wait_gather(j, b)
                    pltpu.sync_copy(
                        rows_b[b],
                        acc_sh.at[dst_v.at[pl.ds(j * _EDGE_BLK, _EDGE_BLK)]],
                        add=True)

                    @pl.when(j + _NBUF < fire_limit)
                    def _():
                        start_gather(j + _NBUF, b)
                return carry

            lax.fori_loop(0, n_iters, body, 0)

        def stage(eblk_off, nblk):
            ne = nblk * _EDGE_BLK
            pltpu.sync_copy(ei_hbm.at[0, pl.ds(eblk_off * _EDGE_BLK, ne)],
                            src_v.at[pl.ds(0, ne)])
            pltpu.sync_copy(ei_hbm.at[1, pl.ds(eblk_off * _EDGE_BLK, ne)],
                            dst_v.at[pl.ds(0, ne)])

        base_blk = wid * _BLK_FULL

        @pl.when(is_last)
        def _():
            stage(31 * _BLK_FULL, _BLK_LAST)

        @pl.when(jnp.logical_not(is_last))
        def _():
            stage(base_blk, _HALF)

        n0 = jnp.where(is_last, _BLK_LAST // _NBUF, _HALF // _NBUF)
        lim0 = jnp.where(is_last, _BLK_LAST, _HALF)
        run_pipeline(n0, lim0)

        @pl.when(jnp.logical_not(is_last))
        def _():
            stage(base_blk + _HALF, _HALF)
            run_pipeline(_HALF // _NBUF, _HALF)

        plsc.subcore_barrier()

        out_hbm = (out0_hbm, out1_hbm)
        for ci in range(2):
            @pl.when((c == ci) & (s < 15))
            def _(ci=ci):
                rows = pl.ds(s * _RPS, _RPS)
                pltpu.sync_copy(acc_sh.at[rows], out_hbm[ci].at[rows])

            @pl.when((c == ci) & (s == 15))
            def _(ci=ci):
                rows = pl.ds(15 * _RPS, _LAST_ROWS)
                pltpu.sync_copy(acc_sh.at[rows], out_hbm[ci].at[rows])

    return k(h, edge_index, zeros_blk)


def _tc_layer(h, a0, a1, w1, b1, w2, b2, gamma, beta):
    def body(h_ref, a0_ref, a1_ref, w1_ref, b1_ref, w2_ref, b2_ref,
             g_ref, bt_ref, o_ref):
        hin = h_ref[...] + a0_ref[...] + a1_ref[...]
        z = jax.lax.dot_general(hin, w1_ref[...], (((1,), (0,)), ((), ())),
                                preferred_element_type=jnp.float32)
        z = jnp.maximum(z + b1_ref[...], 0.0)
        o = jax.lax.dot_general(z, w2_ref[...], (((1,), (0,)), ((), ())),
                                preferred_element_type=jnp.float32)
        o = (o + b2_ref[...]) * (g_ref[...] * _INV) + bt_ref[...]
        o_ref[...] = jnp.maximum(o, 0.0)

    row_spec = pl.BlockSpec((_TC_BLK, D), lambda i: (i, 0))
    full = lambda shape: pl.BlockSpec(shape, lambda i: (0,) * len(shape))
    return pl.pallas_call(
        body,
        grid=(_TC_GRID,),
        in_specs=[row_spec, row_spec, row_spec,
                  full((D, H)), full((1, H)), full((H, H)), full((1, H)),
                  full((1, H)), full((1, H))],
        out_specs=row_spec,
        out_shape=jax.ShapeDtypeStruct((N, H), jnp.float32),
    )(h, a0, a1, w1, b1, w2, b2, gamma, beta)


def _tc_pool_head(h, batch3d, fc1_w, fc1_b, fc2_w, fc2_b):
    def body(h_ref, b_ref, w1_ref, b1_ref, w2_ref, b2_ref, o_ref, acc, cnt):
        i = pl.program_id(0)

        @pl.when(i == 0)
        def _():
            acc[...] = jnp.zeros_like(acc)
            cnt[...] = jnp.zeros_like(cnt)

        bm = b_ref[0]
        gids = jax.lax.broadcasted_iota(jnp.int32, (G, _TC_BLK), 0)
        onehot_t = (gids == bm).astype(jnp.float32)
        acc[...] += jax.lax.dot_general(
            onehot_t, h_ref[...], (((1,), (0,)), ((), ())),
            preferred_element_type=jnp.float32)
        cnt[...] += jnp.sum(onehot_t, axis=1, keepdims=True)

        @pl.when(i == _TC_GRID - 1)
        def _():
            pooled = acc[...] / jnp.maximum(cnt[...], 1.0)
            z = jax.lax.dot_general(pooled, w1_ref[...], (((1,), (0,)), ((), ())),
                                    preferred_element_type=jnp.float32)
            z = jnp.maximum(z + b1_ref[...], 0.0)
            o = jax.lax.dot_general(z, w2_ref[...], (((1,), (0,)), ((), ())),
                                    preferred_element_type=jnp.float32)
            o_ref[...] = o + b2_ref[...]

    full = lambda shape: pl.BlockSpec(shape, lambda i: (0,) * len(shape))
    return pl.pallas_call(
        body,
        grid=(_TC_GRID,),
        in_specs=[pl.BlockSpec((_TC_BLK, D), lambda i: (i, 0)),
                  pl.BlockSpec((1, 1, _TC_BLK), lambda i: (i, 0, 0)),
                  full((H, G)), full((1, G)), full((G, 1)), full((1, 1))],
        out_specs=full((G, 1)),
        out_shape=jax.ShapeDtypeStruct((G, 1), jnp.float32),
        scratch_shapes=[pltpu.VMEM((G, D), jnp.float32),
                        pltpu.VMEM((G, 1), jnp.float32)],
        compiler_params=pltpu.CompilerParams(
            dimension_semantics=("arbitrary",)),
    )(h, batch3d, fc1_w, fc1_b, fc2_w, fc2_b)


def kernel(x, edge_index, batch,
           l0_w1, l0_b1, l0_w2, l0_b2, l0_gamma, l0_beta,
           l1_w1, l1_b1, l1_w2, l1_b2, l1_gamma, l1_beta,
           l2_w1, l2_b1, l2_w2, l2_b2, l2_gamma, l2_beta,
           fc1_w, fc1_b, fc2_w, fc2_b):
    batch3d = batch.reshape(_TC_GRID, 1, _TC_BLK)
    zeros_blk = jnp.zeros((_RPS, D), jnp.float32)

    h = x
    params = [
        (l0_w1, l0_b1, l0_w2, l0_b2, l0_gamma, l0_beta),
        (l1_w1, l1_b1, l1_w2, l1_b2, l1_gamma, l1_beta),
        (l2_w1, l2_b1, l2_w2, l2_b2, l2_gamma, l2_beta),
    ]
    for w1, b1, w2, b2, g, b in params:
        agg0, agg1 = _sc_aggregate(h, edge_index, zeros_blk)
        h = _tc_layer(h, agg0, agg1, w1, b1.reshape(1, H),
                      w2, b2.reshape(1, H), g.reshape(1, H), b.reshape(1, H))

    out = _tc_pool_head(h, batch3d, fc1_w, fc1_b.reshape(1, G),
                        fc2_w, fc2_b.reshape(1, 1))
    return jnp.squeeze(out, axis=-1)


# speedup vs baseline: 1.1106x; 1.0805x over previous
import functools

import jax
import jax.numpy as jnp
from jax import lax
from jax.experimental import pallas as pl
from jax.experimental.pallas import tpu as pltpu
from jax.experimental.pallas import tpu_sc as plsc

N = 10000
E = 320000
D = 128
H = 128
G = 64

_INV = 0.9999950000374996

_EDGE_BLK = 128
_NBLOCKS = E // _EDGE_BLK
_NTILES = 32
_BLK_FULL = 80
_BLK_LAST = _NBLOCKS - 31 * _BLK_FULL
_HALF = _BLK_FULL // 2
_NBUF = 2

_RPS = 640
_LAST_ROWS = N - 15 * _RPS

_TC_BLK = 1000
_TC_GRID = N // _TC_BLK


def _sc_aggregate(h, edge_index, zeros_blk):
    mesh = plsc.VectorSubcoreMesh(core_axis_name="c", subcore_axis_name="s")

    @functools.partial(
        pl.kernel,
        mesh=mesh,
        out_type=(jax.ShapeDtypeStruct((N, D), jnp.float32),
                  jax.ShapeDtypeStruct((N, D), jnp.float32)),
        scratch_types=[
            pltpu.VMEM((_HALF * _EDGE_BLK,), jnp.int32),
            pltpu.VMEM((_HALF * _EDGE_BLK,), jnp.int32),
            pltpu.VMEM((_EDGE_BLK, D), jnp.float32),
            pltpu.VMEM((_EDGE_BLK, D), jnp.float32),
            pltpu.VMEM_SHARED((N, D), jnp.float32),
            pltpu.SemaphoreType.DMA,
            pltpu.SemaphoreType.DMA,
        ],
    )
    def k(h_hbm, ei_hbm, zeros_hbm, out0_hbm, out1_hbm,
          src_v, dst_v, rows_v0, rows_v1, acc_sh, gsem0, gsem1):
        rows_b = (rows_v0, rows_v1)
        gsem_b = (gsem0, gsem1)
        c = lax.axis_index("c")
        s = lax.axis_index("s")
        wid = c * 16 + s
        is_last = wid == _NTILES - 1

        @pl.when(s < 15)
        def _():
            pltpu.sync_copy(zeros_hbm, acc_sh.at[pl.ds(s * _RPS, _RPS)])

        @pl.when(s == 15)
        def _():
            pltpu.sync_copy(zeros_hbm.at[pl.ds(0, _LAST_ROWS)],
                            acc_sh.at[pl.ds(15 * _RPS, _LAST_ROWS)])

        plsc.subcore_barrier()

        def start_gather(j, b):
            pltpu.async_copy(h_hbm.at[src_v.at[pl.ds(j * _EDGE_BLK, _EDGE_BLK)]],
                             rows_b[b], gsem_b[b])

        def wait_gather(j, b):
            pltpu.make_async_copy(
                h_hbm.at[src_v.at[pl.ds(j * _EDGE_BLK, _EDGE_BLK)]],
                rows_b[b], gsem_b[b]).wait()

        def run_pipeline(n_iters, fire_limit):
            for b in range(_NBUF):
                start_gather(b, b)

            def body(g, carry):
                for b in range(_NBUF):
                    j = g * _NBUF + b
                    wait_gather(j, b)
                    pltpu.sync_copy(
                        rows_b[b],
                        acc_sh.at[dst_v.at[pl.ds(j * _EDGE_BLK, _EDGE_BLK)]],
                        add=True)

                    @pl.when(j + _NBUF < fire_limit)
                    def _():
                        start_gather(j + _NBUF, b)
                return carry

            lax.fori_loop(0, n_iters, body, 0)

        def stage(eblk_off, nblk):
            ne = nblk * _EDGE_BLK
            pltpu.sync_copy(ei_hbm.at[0, pl.ds(eblk_off * _EDGE_BLK, ne)],
                            src_v.at[pl.ds(0, ne)])
            pltpu.sync_copy(ei_hbm.at[1, pl.ds(eblk_off * _EDGE_BLK, ne)],
                            dst_v.at[pl.ds(0, ne)])

        base_blk = wid * _BLK_FULL

        @pl.when(is_last)
        def _():
            stage(31 * _BLK_FULL, _BLK_LAST)

        @pl.when(jnp.logical_not(is_last))
        def _():
            stage(base_blk, _HALF)

        n0 = jnp.where(is_last, _BLK_LAST // _NBUF, _HALF // _NBUF)
        lim0 = jnp.where(is_last, _BLK_LAST, _HALF)
        run_pipeline(n0, lim0)

        @pl.when(jnp.logical_not(is_last))
        def _():
            stage(base_blk + _HALF, _HALF)
            run_pipeline(_HALF // _NBUF, _HALF)

        plsc.subcore_barrier()

        out_hbm = (out0_hbm, out1_hbm)
        for ci in range(2):
            @pl.when((c == ci) & (s < 15))
            def _(ci=ci):
                rows = pl.ds(s * _RPS, _RPS)
                pltpu.sync_copy(acc_sh.at[rows], out_hbm[ci].at[rows])

            @pl.when((c == ci) & (s == 15))
            def _(ci=ci):
                rows = pl.ds(15 * _RPS, _LAST_ROWS)
                pltpu.sync_copy(acc_sh.at[rows], out_hbm[ci].at[rows])

    return k(h, edge_index, zeros_blk)


def _tc_layer(h, a0, a1, w1, b1, w2, b2, gamma, beta):
    def body(h_ref, a0_ref, a1_ref, w1_ref, b1_ref, w2_ref, b2_ref,
             g_ref, bt_ref, o_ref):
        hin = h_ref[...] + a0_ref[...] + a1_ref[...]
        z = jax.lax.dot_general(hin, w1_ref[...], (((1,), (0,)), ((), ())),
                                preferred_element_type=jnp.float32)
        z = jnp.maximum(z + b1_ref[...], 0.0)
        o = jax.lax.dot_general(z, w2_ref[...], (((1,), (0,)), ((), ())),
                                preferred_element_type=jnp.float32)
        o = (o + b2_ref[...]) * (g_ref[...] * _INV) + bt_ref[...]
        o_ref[...] = jnp.maximum(o, 0.0)

    row_spec = pl.BlockSpec((_TC_BLK, D), lambda i: (i, 0))
    full = lambda shape: pl.BlockSpec(shape, lambda i: (0,) * len(shape))
    return pl.pallas_call(
        body,
        grid=(_TC_GRID,),
        in_specs=[row_spec, row_spec, row_spec,
                  full((D, H)), full((1, H)), full((H, H)), full((1, H)),
                  full((1, H)), full((1, H))],
        out_specs=row_spec,
        out_shape=jax.ShapeDtypeStruct((N, H), jnp.float32),
    )(h, a0, a1, w1, b1, w2, b2, gamma, beta)


def _tc_pool_head(h, batch3d, fc1_w, fc1_b, fc2_w, fc2_b):
    def body(h_ref, b_ref, w1_ref, b1_ref, w2_ref, b2_ref, o_ref, acc, cnt):
        i = pl.program_id(0)

        @pl.when(i == 0)
        def _():
            acc[...] = jnp.zeros_like(acc)
            cnt[...] = jnp.zeros_like(cnt)

        bm = b_ref[0]
        gids = jax.lax.broadcasted_iota(jnp.int32, (G, _TC_BLK), 0)
        onehot_t = (gids == bm).astype(jnp.float32)
        acc[...] += jax.lax.dot_general(
            onehot_t, h_ref[...], (((1,), (0,)), ((), ())),
            preferred_element_type=jnp.float32)
        cnt[...] += jnp.sum(onehot_t, axis=1, keepdims=True)

        @pl.when(i == _TC_GRID - 1)
        def _():
            pooled = acc[...] / jnp.maximum(cnt[...], 1.0)
            z = jax.lax.dot_general(pooled, w1_ref[...], (((1,), (0,)), ((), ())),
                                    preferred_element_type=jnp.float32)
            z = jnp.maximum(z + b1_ref[...], 0.0)
            o = jax.lax.dot_general(z, w2_ref[...], (((1,), (0,)), ((), ())),
                                    preferred_element_type=jnp.float32)
            o_ref[...] = o + b2_ref[...]

    full = lambda shape: pl.BlockSpec(shape, lambda i: (0,) * len(shape))
    return pl.pallas_call(
        body,
        grid=(_TC_GRID,),
        in_specs=[pl.BlockSpec((_TC_BLK, D), lambda i: (i, 0)),
                  pl.BlockSpec((1, 1, _TC_BLK), lambda i: (i, 0, 0)),
                  full((H, G)), full((1, G)), full((G, 1)), full((1, 1))],
        out_specs=full((G, 1)),
        out_shape=jax.ShapeDtypeStruct((G, 1), jnp.float32),
        scratch_shapes=[pltpu.VMEM((G, D), jnp.float32),
                        pltpu.VMEM((G, 1), jnp.float32)],
        compiler_params=pltpu.CompilerParams(
            dimension_semantics=("arbitrary",)),
    )(h, batch3d, fc1_w, fc1_b, fc2_w, fc2_b)


def kernel(x, edge_index, batch,
           l0_w1, l0_b1, l0_w2, l0_b2, l0_gamma, l0_beta,
           l1_w1, l1_b1, l1_w2, l1_b2, l1_gamma, l1_beta,
           l2_w1, l2_b1, l2_w2, l2_b2, l2_gamma, l2_beta,
           fc1_w, fc1_b, fc2_w, fc2_b):
    batch3d = batch.reshape(_TC_GRID, 1, _TC_BLK)
    zeros_blk = jnp.zeros((_RPS, D), jnp.float32)

    h = x
    params = [
        (l0_w1, l0_b1, l0_w2, l0_b2, l0_gamma, l0_beta),
        (l1_w1, l1_b1, l1_w2, l1_b2, l1_gamma, l1_beta),
        (l2_w1, l2_b1, l2_w2, l2_b2, l2_gamma, l2_beta),
    ]
    for w1, b1, w2, b2, g, b in params:
        agg0, agg1 = _sc_aggregate(h, edge_index, zeros_blk)
        h = _tc_layer(h, agg0, agg1, w1, b1.reshape(1, H),
                      w2, b2.reshape(1, H), g.reshape(1, H), b.reshape(1, H))

    out = _tc_pool_head(h, batch3d, fc1_w, fc1_b.reshape(1, G),
                        fc2_w, fc2_b.reshape(1, 1))
    return jnp.squeeze(out, axis=-1)


# speedup vs baseline: 1.1324x; 1.0196x over previous
import functools

import jax
import jax.numpy as jnp
from jax import lax
from jax.experimental import pallas as pl
from jax.experimental.pallas import tpu as pltpu
from jax.experimental.pallas import tpu_sc as plsc

N = 10000
E = 320000
D = 128
H = 128
G = 64

_INV = 0.9999950000374996

_EDGE_BLK = 128
_NBLOCKS = E // _EDGE_BLK
_NTILES = 32
_BLK_FULL = 80
_BLK_LAST = _NBLOCKS - 31 * _BLK_FULL
_HALF = _BLK_FULL // 2
_NBUF = 2

_RPS = 640
_LAST_ROWS = N - 15 * _RPS

_TC_BLK = 1000
_TC_GRID = N // _TC_BLK


def _sc_aggregate(h, edge_index, zeros_blk):
    mesh = plsc.VectorSubcoreMesh(core_axis_name="c", subcore_axis_name="s")

    @functools.partial(
        pl.kernel,
        mesh=mesh,
        out_type=(jax.ShapeDtypeStruct((N, D), jnp.float32),
                  jax.ShapeDtypeStruct((N, D), jnp.float32)),
        scratch_types=[
            pltpu.VMEM((_HALF * _EDGE_BLK,), jnp.int32),
            pltpu.VMEM((_HALF * _EDGE_BLK,), jnp.int32),
            pltpu.VMEM((_EDGE_BLK, D), jnp.float32),
            pltpu.VMEM((_EDGE_BLK, D), jnp.float32),
            pltpu.VMEM_SHARED((N, D), jnp.float32),
            pltpu.SemaphoreType.DMA,
            pltpu.SemaphoreType.DMA,
        ],
    )
    def k(h_hbm, ei_hbm, zeros_hbm, out0_hbm, out1_hbm,
          src_v, dst_v, rows_v0, rows_v1, acc_sh, gsem0, gsem1):
        rows_b = (rows_v0, rows_v1)
        gsem_b = (gsem0, gsem1)
        c = lax.axis_index("c")
        s = lax.axis_index("s")
        wid = c * 16 + s
        is_last = wid == _NTILES - 1

        @pl.when(s < 15)
        def _():
            pltpu.async_copy(zeros_hbm, acc_sh.at[pl.ds(s * _RPS, _RPS)],
                             gsem0)

        @pl.when(s == 15)
        def _():
            pltpu.async_copy(zeros_hbm.at[pl.ds(0, _LAST_ROWS)],
                             acc_sh.at[pl.ds(15 * _RPS, _LAST_ROWS)], gsem0)

        def start_gather(j, b):
            pltpu.async_copy(h_hbm.at[src_v.at[pl.ds(j * _EDGE_BLK, _EDGE_BLK)]],
                             rows_b[b], gsem_b[b])

        def wait_gather(j, b):
            pltpu.make_async_copy(
                h_hbm.at[src_v.at[pl.ds(j * _EDGE_BLK, _EDGE_BLK)]],
                rows_b[b], gsem_b[b]).wait()

        def run_pipeline(n_iters, fire_limit):
            for b in range(_NBUF):
                start_gather(b, b)

            def body(g, carry):
                for b in range(_NBUF):
                    j = g * _NBUF + b
                    wait_gather(j, b)
                    pltpu.sync_copy(
                        rows_b[b],
                        acc_sh.at[dst_v.at[pl.ds(j * _EDGE_BLK, _EDGE_BLK)]],
                        add=True)

                    @pl.when(j + _NBUF < fire_limit)
                    def _():
                        start_gather(j + _NBUF, b)
                return carry

            lax.fori_loop(0, n_iters, body, 0)

        def stage_async(eblk_off, nblk):
            ne = nblk * _EDGE_BLK
            pltpu.async_copy(ei_hbm.at[0, pl.ds(eblk_off * _EDGE_BLK, ne)],
                             src_v.at[pl.ds(0, ne)], gsem1)
            pltpu.async_copy(ei_hbm.at[1, pl.ds(eblk_off * _EDGE_BLK, ne)],
                             dst_v.at[pl.ds(0, ne)], gsem0)

        def stage_wait(eblk_off, nblk):
            ne = nblk * _EDGE_BLK
            pltpu.make_async_copy(
                ei_hbm.at[0, pl.ds(eblk_off * _EDGE_BLK, ne)],
                src_v.at[pl.ds(0, ne)], gsem1).wait()
            pltpu.make_async_copy(
                ei_hbm.at[1, pl.ds(eblk_off * _EDGE_BLK, ne)],
                dst_v.at[pl.ds(0, ne)], gsem0).wait()

        base_blk = wid * _BLK_FULL

        @pl.when(is_last)
        def _():
            stage_async(31 * _BLK_FULL, _BLK_LAST)

        @pl.when(jnp.logical_not(is_last))
        def _():
            stage_async(base_blk, _HALF)

        @pl.when(s < 15)
        def _():
            pltpu.make_async_copy(zeros_hbm, acc_sh.at[pl.ds(s * _RPS, _RPS)],
                                  gsem0).wait()

        @pl.when(s == 15)
        def _():
            pltpu.make_async_copy(
                zeros_hbm.at[pl.ds(0, _LAST_ROWS)],
                acc_sh.at[pl.ds(15 * _RPS, _LAST_ROWS)], gsem0).wait()

        plsc.subcore_barrier()

        @pl.when(is_last)
        def _():
            stage_wait(31 * _BLK_FULL, _BLK_LAST)

        @pl.when(jnp.logical_not(is_last))
        def _():
            stage_wait(base_blk, _HALF)

        n0 = jnp.where(is_last, _BLK_LAST // _NBUF, _HALF // _NBUF)
        lim0 = jnp.where(is_last, _BLK_LAST, _HALF)
        run_pipeline(n0, lim0)

        @pl.when(jnp.logical_not(is_last))
        def _():
            stage_async(base_blk + _HALF, _HALF)
            stage_wait(base_blk + _HALF, _HALF)
            run_pipeline(_HALF // _NBUF, _HALF)

        plsc.subcore_barrier()

        out_hbm = (out0_hbm, out1_hbm)
        for ci in range(2):
            @pl.when((c == ci) & (s < 15))
            def _(ci=ci):
                rows = pl.ds(s * _RPS, _RPS)
                pltpu.sync_copy(acc_sh.at[rows], out_hbm[ci].at[rows])

            @pl.when((c == ci) & (s == 15))
            def _(ci=ci):
                rows = pl.ds(15 * _RPS, _LAST_ROWS)
                pltpu.sync_copy(acc_sh.at[rows], out_hbm[ci].at[rows])

    return k(h, edge_index, zeros_blk)


def _tc_layer(h, a0, a1, w1, b1, w2, b2, gamma, beta):
    def body(h_ref, a0_ref, a1_ref, w1_ref, b1_ref, w2_ref, b2_ref,
             g_ref, bt_ref, o_ref):
        hin = h_ref[...] + a0_ref[...] + a1_ref[...]
        z = jax.lax.dot_general(hin, w1_ref[...], (((1,), (0,)), ((), ())),
                                preferred_element_type=jnp.float32)
        z = jnp.maximum(z + b1_ref[...], 0.0)
        o = jax.lax.dot_general(z, w2_ref[...], (((1,), (0,)), ((), ())),
                                preferred_element_type=jnp.float32)
        o = (o + b2_ref[...]) * (g_ref[...] * _INV) + bt_ref[...]
        o_ref[...] = jnp.maximum(o, 0.0)

    row_spec = pl.BlockSpec((_TC_BLK, D), lambda i: (i, 0))
    full = lambda shape: pl.BlockSpec(shape, lambda i: (0,) * len(shape))
    return pl.pallas_call(
        body,
        grid=(_TC_GRID,),
        in_specs=[row_spec, row_spec, row_spec,
                  full((D, H)), full((1, H)), full((H, H)), full((1, H)),
                  full((1, H)), full((1, H))],
        out_specs=row_spec,
        out_shape=jax.ShapeDtypeStruct((N, H), jnp.float32),
    )(h, a0, a1, w1, b1, w2, b2, gamma, beta)


def _tc_pool_head(h, batch3d, fc1_w, fc1_b, fc2_w, fc2_b):
    def body(h_ref, b_ref, w1_ref, b1_ref, w2_ref, b2_ref, o_ref, acc, cnt):
        i = pl.program_id(0)

        @pl.when(i == 0)
        def _():
            acc[...] = jnp.zeros_like(acc)
            cnt[...] = jnp.zeros_like(cnt)

        bm = b_ref[0]
        gids = jax.lax.broadcasted_iota(jnp.int32, (G, _TC_BLK), 0)
        onehot_t = (gids == bm).astype(jnp.float32)
        acc[...] += jax.lax.dot_general(
            onehot_t, h_ref[...], (((1,), (0,)), ((), ())),
            preferred_element_type=jnp.float32)
        cnt[...] += jnp.sum(onehot_t, axis=1, keepdims=True)

        @pl.when(i == _TC_GRID - 1)
        def _():
            pooled = acc[...] / jnp.maximum(cnt[...], 1.0)
            z = jax.lax.dot_general(pooled, w1_ref[...], (((1,), (0,)), ((), ())),
                                    preferred_element_type=jnp.float32)
            z = jnp.maximum(z + b1_ref[...], 0.0)
            o = jax.lax.dot_general(z, w2_ref[...], (((1,), (0,)), ((), ())),
                                    preferred_element_type=jnp.float32)
            o_ref[...] = o + b2_ref[...]

    full = lambda shape: pl.BlockSpec(shape, lambda i: (0,) * len(shape))
    return pl.pallas_call(
        body,
        grid=(_TC_GRID,),
        in_specs=[pl.BlockSpec((_TC_BLK, D), lambda i: (i, 0)),
                  pl.BlockSpec((1, 1, _TC_BLK), lambda i: (i, 0, 0)),
                  full((H, G)), full((1, G)), full((G, 1)), full((1, 1))],
        out_specs=full((G, 1)),
        out_shape=jax.ShapeDtypeStruct((G, 1), jnp.float32),
        scratch_shapes=[pltpu.VMEM((G, D), jnp.float32),
                        pltpu.VMEM((G, 1), jnp.float32)],
        compiler_params=pltpu.CompilerParams(
            dimension_semantics=("arbitrary",)),
    )(h, batch3d, fc1_w, fc1_b, fc2_w, fc2_b)


def kernel(x, edge_index, batch,
           l0_w1, l0_b1, l0_w2, l0_b2, l0_gamma, l0_beta,
           l1_w1, l1_b1, l1_w2, l1_b2, l1_gamma, l1_beta,
           l2_w1, l2_b1, l2_w2, l2_b2, l2_gamma, l2_beta,
           fc1_w, fc1_b, fc2_w, fc2_b):
    batch3d = batch.reshape(_TC_GRID, 1, _TC_BLK)
    zeros_blk = jnp.zeros((_RPS, D), jnp.float32)

    h = x
    params = [
        (l0_w1, l0_b1, l0_w2, l0_b2, l0_gamma, l0_beta),
        (l1_w1, l1_b1, l1_w2, l1_b2, l1_gamma, l1_beta),
        (l2_w1, l2_b1, l2_w2, l2_b2, l2_gamma, l2_beta),
    ]
    for w1, b1, w2, b2, g, b in params:
        agg0, agg1 = _sc_aggregate(h, edge_index, zeros_blk)
        h = _tc_layer(h, agg0, agg1, w1, b1.reshape(1, H),
                      w2, b2.reshape(1, H), g.reshape(1, H), b.reshape(1, H))

    out = _tc_pool_head(h, batch3d, fc1_w, fc1_b.reshape(1, G),
                        fc2_w, fc2_b.reshape(1, 1))
    return jnp.squeeze(out, axis=-1)
